# baseline (speedup 1.0000x reference)
import functools

import jax
import jax.numpy as jnp
from jax import lax
from jax.experimental import pallas as pl
from jax.experimental.pallas import tpu as pltpu
from jax.experimental.pallas import tpu_sc as plsc


def _make_gather(B, H, V, D, NC, NS):
    NW = NC * NS
    TB = B // 128
    TH = H // 8
    TD = D // 8
    NBLK = TH * TB
    BPW = NBLK // NW
    assert NBLK % NW == 0 and BPW >= 2

    mesh = plsc.VectorSubcoreMesh(core_axis_name="c", subcore_axis_name="s")

    @functools.partial(
        pl.kernel,
        mesh=mesh,
        out_type=jax.ShapeDtypeStruct((H, TD, TB, 8, 128), jnp.float32),
        scratch_types=[
            pltpu.VMEM((8, 128), jnp.int32),
            pltpu.VMEM((8, 128), jnp.int32),
            pltpu.VMEM((1024, D), jnp.float32),
            pltpu.VMEM((1024, D), jnp.float32),
            pltpu.VMEM((TD, 8, 128), jnp.float32),
            pltpu.VMEM((TD, 8, 128), jnp.float32),
            pltpu.SemaphoreType.DMA,
            pltpu.SemaphoreType.DMA,
            pltpu.SemaphoreType.DMA,
            pltpu.SemaphoreType.DMA,
            pltpu.SemaphoreType.DMA,
            pltpu.SemaphoreType.DMA,
        ],
        compiler_params=pltpu.CompilerParams(
            use_tc_tiling_on_sc=False, needs_layout_passes=False
        ),
    )
    def gather_kernel(
        ids_hbm, table_hbm, out_hbm,
        idxb0, idxb1, rows0, rows1, tbufa, tbufb,
        isem0, isem1, gsem0, gsem1, osema, osemb,
    ):
        w = lax.axis_index("s") * NC + lax.axis_index("c")
        blk0 = w * BPW
        idxb = (idxb0, idxb1)
        rows = (rows0, rows1)
        isem = (isem0, isem1)
        gsem = (gsem0, gsem1)
        tbuf = (tbufa, tbufb)
        osem = (osema, osemb)
        iota = lax.iota(jnp.int32, 16)

        def idx_src(m):
            blk = blk0 + m
            return ids_hbm.at[blk // TB, blk % TB]

        def stage_idx(m, p):
            pltpu.async_copy(idx_src(m), idxb[p], isem[p])

        def wait_idx(m, p):
            pltpu.make_async_copy(idx_src(m), idxb[p], isem[p]).wait()

        def fire_gathers(p):
            for s in range(8):
                pltpu.async_copy(
                    table_hbm.at[idxb[p].at[s]],
                    rows[p].at[pl.ds(s * 128, 128)],
                    gsem[p],
                )

        def wait_gathers(p):
            for s in range(8):
                pltpu.make_async_copy(
                    table_hbm.at[idxb[p].at[s]],
                    rows[p].at[pl.ds(s * 128, 128)],
                    gsem[p],
                ).wait()

        def transpose_chunk(p, c, x):
            base = c * 128
            for d in range(D):
                t_d, s_d = divmod(d, 8)
                col = lax.full((16,), d, jnp.int32)
                for k in range(8):
                    vec = plsc.load_gather(
                        rows[p], [base + 16 * k + iota, col]
                    )
                    tbuf[x][t_d, s_d, pl.ds(16 * k, 16)] = vec

        def fire_outs(m, c, x):
            blk = blk0 + m
            h = (blk // TB) * 8 + c
            t_b = blk % TB
            for t_d in range(TD):
                pltpu.async_copy(
                    tbuf[x].at[t_d], out_hbm.at[h, t_d, t_b], osem[x]
                )

        def wait_outs(x):
            for t_d in range(TD):
                pltpu.make_async_copy(
                    tbuf[x].at[t_d], out_hbm.at[0, t_d, 0], osem[x]
                ).wait()

        def process_block(m, p):
            q = 1 - p

            @pl.when(m < BPW - 1)
            def _():
                wait_idx(m + 1, q)
                fire_gathers(q)

            wait_gathers(p)

            def chunk_body(c, carry):
                def do(x):
                    @pl.when((m > 0) | (c >= 2))
                    def _():
                        wait_outs(x)

                    transpose_chunk(p, c, x)
                    fire_outs(m, c, x)

                @pl.when(c % 2 == 0)
                def _():
                    do(0)

                @pl.when(c % 2 == 1)
                def _():
                    do(1)

                return carry

            lax.fori_loop(0, 8, chunk_body, 0)

            @pl.when(m < BPW - 2)
            def _():
                stage_idx(m + 2, p)

        pltpu.sync_copy(idx_src(0), idxb[0])
        fire_gathers(0)
        stage_idx(1, 1)

        def block_body(m, carry):
            @pl.when(m % 2 == 0)
            def _():
                process_block(m, 0)

            @pl.when(m % 2 == 1)
            def _():
                process_block(m, 1)

            return carry

        lax.fori_loop(0, BPW, block_body, 0)
        wait_outs(0)
        wait_outs(1)

    return gather_kernel


def kernel(input_ids, table):
    B, H = input_ids.shape
    V, D = table.shape
    info = plsc.get_sparse_core_info()
    NC, NS = info.num_cores, info.num_subcores
    ids4 = input_ids.T.reshape(H // 8, 8, B // 128, 128).transpose(0, 2, 1, 3)
    gather = _make_gather(B, H, V, D, NC, NS)
    out5 = gather(ids4, table)
    return out5.transpose(2, 4, 0, 1, 3).reshape(B, H, D)


# speedup vs baseline: 2.2954x; 2.2954x over previous
import functools

import jax
import jax.numpy as jnp
from jax import lax
from jax.experimental import pallas as pl
from jax.experimental.pallas import tpu as pltpu
from jax.experimental.pallas import tpu_sc as plsc


def _make_gather(B, H, V, D, NC, NS):
    NW = NC * NS
    TB = B // 128
    TH = H // 8
    TD = D // 8
    NBLK = TH * TB
    BPW = NBLK // NW
    assert NBLK % NW == 0 and BPW >= 2

    mesh = plsc.VectorSubcoreMesh(core_axis_name="c", subcore_axis_name="s")

    @functools.partial(
        pl.kernel,
        mesh=mesh,
        out_type=jax.ShapeDtypeStruct((H, TD, TB, 8, 128), jnp.float32),
        scratch_types=[
            pltpu.VMEM((8, 128), jnp.int32),
            pltpu.VMEM((8, 128), jnp.int32),
            pltpu.VMEM((1024, D), jnp.float32),
            pltpu.VMEM((1024, D), jnp.float32),
            pltpu.VMEM((TD, 8, 129), jnp.float32),
            pltpu.VMEM((TD, 8, 129), jnp.float32),
            pltpu.SemaphoreType.DMA,
            pltpu.SemaphoreType.DMA,
            pltpu.SemaphoreType.DMA,
            pltpu.SemaphoreType.DMA,
            pltpu.SemaphoreType.DMA,
            pltpu.SemaphoreType.DMA,
        ],
        compiler_params=pltpu.CompilerParams(
            use_tc_tiling_on_sc=False, needs_layout_passes=False
        ),
    )
    def gather_kernel(
        ids_hbm, table_hbm, out_hbm,
        idxb0, idxb1, rows0, rows1, tbufa, tbufb,
        isem0, isem1, gsem0, gsem1, osema, osemb,
    ):
        w = lax.axis_index("s") * NC + lax.axis_index("c")
        blk0 = w * BPW
        idxb = (idxb0, idxb1)
        rows = (rows0, rows1)
        isem = (isem0, isem1)
        gsem = (gsem0, gsem1)
        tbuf = (tbufa, tbufb)
        osem = (osema, osemb)
        iota = lax.iota(jnp.int32, 16)

        def idx_src(m):
            blk = blk0 + m
            return ids_hbm.at[blk // TB, blk % TB]

        def stage_idx(m, p):
            pltpu.async_copy(idx_src(m), idxb[p], isem[p])

        def wait_idx(m, p):
            pltpu.make_async_copy(idx_src(m), idxb[p], isem[p]).wait()

        def fire_gathers(p):
            for s in range(8):
                pltpu.async_copy(
                    table_hbm.at[idxb[p].at[s]],
                    rows[p].at[pl.ds(s * 128, 128)],
                    gsem[p],
                )

        def wait_gathers(p):
            for s in range(8):
                pltpu.make_async_copy(
                    table_hbm.at[idxb[p].at[s]],
                    rows[p].at[pl.ds(s * 128, 128)],
                    gsem[p],
                ).wait()

        tj = []
        for j in range(D // 16):
            dvec = 16 * j + iota
            tj.append((dvec // 8, dvec % 8))

        def transpose_chunk(p, c, x):
            base = c * 128

            def body(l, carry):
                lcol = lax.broadcast(l, (16,))
                for j in range(D // 16):
                    vec = rows[p][base + l, pl.ds(16 * j, 16)]
                    plsc.store_scatter(tbuf[x], [tj[j][0], tj[j][1], lcol], vec)
                return carry

            lax.fori_loop(0, 128, body, 0, unroll=4)

        def fire_outs(m, c, x):
            blk = blk0 + m
            h = (blk // TB) * 8 + c
            t_b = blk % TB
            pltpu.async_copy(
                tbuf[x].at[:, :, pl.ds(0, 128)], out_hbm.at[h, :, t_b], osem[x]
            )

        def wait_outs(x):
            pltpu.make_async_copy(
                tbuf[x].at[:, :, pl.ds(0, 128)], out_hbm.at[0, :, 0], osem[x]
            ).wait()

        def process_block(m, p):
            q = 1 - p

            @pl.when(m < BPW - 1)
            def _():
                wait_idx(m + 1, q)
                fire_gathers(q)

            wait_gathers(p)

            def chunk_body(c, carry):
                def do(x):
                    @pl.when((m > 0) | (c >= 2))
                    def _():
                        wait_outs(x)

                    transpose_chunk(p, c, x)
                    fire_outs(m, c, x)

                @pl.when(c % 2 == 0)
                def _():
                    do(0)

                @pl.when(c % 2 == 1)
                def _():
                    do(1)

                return carry

            lax.fori_loop(0, 8, chunk_body, 0)

            @pl.when(m < BPW - 2)
            def _():
                stage_idx(m + 2, p)

        pltpu.sync_copy(idx_src(0), idxb[0])
        fire_gathers(0)
        stage_idx(1, 1)

        def block_body(m, carry):
            @pl.when(m % 2 == 0)
            def _():
                process_block(m, 0)

            @pl.when(m % 2 == 1)
            def _():
                process_block(m, 1)

            return carry

        lax.fori_loop(0, BPW, block_body, 0)
        wait_outs(0)
        wait_outs(1)

    return gather_kernel


def kernel(input_ids, table):
    B, H = input_ids.shape
    V, D = table.shape
    info = plsc.get_sparse_core_info()
    NC, NS = info.num_cores, info.num_subcores
    ids4 = input_ids.T.reshape(H // 8, 8, B // 128, 128).transpose(0, 2, 1, 3)
    gather = _make_gather(B, H, V, D, NC, NS)
    out5 = gather(ids4, table)
    return out5.transpose(2, 4, 0, 1, 3).reshape(B, H, D)


# speedup vs baseline: 3.5160x; 1.5317x over previous
import functools

import jax
import jax.numpy as jnp
from jax import lax
from jax.experimental import pallas as pl
from jax.experimental.pallas import tpu as pltpu
from jax.experimental.pallas import tpu_sc as plsc


def _make_gather(B, H, V, D, NC, NS):
    NW = NC * NS
    TB = B // 128
    TH = H // 8
    TD = D // 8
    NBLK = TH * TB
    BPW = NBLK // NW
    assert NBLK % NW == 0 and BPW >= 2

    mesh = plsc.VectorSubcoreMesh(core_axis_name="c", subcore_axis_name="s")

    @functools.partial(
        pl.kernel,
        mesh=mesh,
        out_type=jax.ShapeDtypeStruct((H, TD, TB, 8, 128), jnp.float32),
        scratch_types=[
            pltpu.VMEM((8, 128), jnp.int32),
            pltpu.VMEM((8, 128), jnp.int32),
            pltpu.VMEM((1024, D), jnp.float32),
            pltpu.VMEM((1024, D), jnp.float32),
            pltpu.VMEM((TD * 8, 129), jnp.float32),
            pltpu.VMEM((TD * 8, 129), jnp.float32),
            pltpu.SemaphoreType.DMA,
            pltpu.SemaphoreType.DMA,
            pltpu.SemaphoreType.DMA,
            pltpu.SemaphoreType.DMA,
            pltpu.SemaphoreType.DMA,
            pltpu.SemaphoreType.DMA,
        ],
        compiler_params=pltpu.CompilerParams(
            use_tc_tiling_on_sc=False, needs_layout_passes=False
        ),
    )
    def gather_kernel(
        ids_hbm, table_hbm, out_hbm,
        idxb0, idxb1, rows0, rows1, tbufa, tbufb,
        isem0, isem1, gsem0, gsem1, osema, osemb,
    ):
        w = lax.axis_index("s") * NC + lax.axis_index("c")
        blk0 = w * BPW
        idxb = (idxb0, idxb1)
        rows = (rows0, rows1)
        isem = (isem0, isem1)
        gsem = (gsem0, gsem1)
        tbuf = (tbufa, tbufb)
        osem = (osema, osemb)
        iota = lax.iota(jnp.int32, 16)

        def idx_src(m):
            blk = blk0 + m
            return ids_hbm.at[blk // TB, blk % TB]

        def stage_idx(m, p):
            pltpu.async_copy(idx_src(m), idxb[p], isem[p])

        def wait_idx(m, p):
            pltpu.make_async_copy(idx_src(m), idxb[p], isem[p]).wait()

        def fire_gathers(p):
            for s in range(8):
                pltpu.async_copy(
                    table_hbm.at[idxb[p].at[s]],
                    rows[p].at[pl.ds(s * 128, 128)],
                    gsem[p],
                )

        def wait_gathers(p):
            for s in range(8):
                pltpu.make_async_copy(
                    table_hbm.at[idxb[p].at[s]],
                    rows[p].at[pl.ds(s * 128, 128)],
                    gsem[p],
                ).wait()

        dv = [16 * j + iota for j in range(D // 16)]

        def transpose_chunk(p, c, x):
            base = c * 128

            def body(g, carry):
                l0 = 4 * g
                vecs = []
                for u in range(4):
                    for j in range(D // 16):
                        vecs.append(rows[p][base + l0 + u, pl.ds(16 * j, 16)])
                for u in range(4):
                    lcol = lax.broadcast(l0 + u, (16,))
                    for j in range(D // 16):
                        plsc.store_scatter(
                            tbuf[x], [dv[j], lcol], vecs[u * (D // 16) + j]
                        )
                return carry

            lax.fori_loop(0, 32, body, 0)

        def fire_outs(m, c, x):
            blk = blk0 + m
            h = (blk // TB) * 8 + c
            t_b = blk % TB
            for t in range(TD):
                pltpu.async_copy(
                    tbuf[x].at[pl.ds(t * 8, 8), pl.ds(0, 128)],
                    out_hbm.at[h, t, t_b],
                    osem[x],
                )

        def wait_outs(x):
            for t in range(TD):
                pltpu.make_async_copy(
                    tbuf[x].at[pl.ds(t * 8, 8), pl.ds(0, 128)],
                    out_hbm.at[0, 0, 0],
                    osem[x],
                ).wait()

        def process_block(m, p):
            q = 1 - p

            @pl.when(m < BPW - 1)
            def _():
                wait_idx(m + 1, q)
                fire_gathers(q)

            wait_gathers(p)

            def chunk_body(c, carry):
                def do(x):
                    @pl.when((m > 0) | (c >= 2))
                    def _():
                        wait_outs(x)

                    transpose_chunk(p, c, x)
                    fire_outs(m, c, x)

                @pl.when(c % 2 == 0)
                def _():
                    do(0)

                @pl.when(c % 2 == 1)
                def _():
                    do(1)

                return carry

            lax.fori_loop(0, 8, chunk_body, 0)

            @pl.when(m < BPW - 2)
            def _():
                stage_idx(m + 2, p)

        pltpu.sync_copy(idx_src(0), idxb[0])
        fire_gathers(0)
        stage_idx(1, 1)

        def block_body(m, carry):
            @pl.when(m % 2 == 0)
            def _():
                process_block(m, 0)

            @pl.when(m % 2 == 1)
            def _():
                process_block(m, 1)

            return carry

        lax.fori_loop(0, BPW, block_body, 0)
        wait_outs(0)
        wait_outs(1)

    return gather_kernel


def kernel(input_ids, table):
    B, H = input_ids.shape
    V, D = table.shape
    info = plsc.get_sparse_core_info()
    NC, NS = info.num_cores, info.num_subcores
    ids4 = input_ids.T.reshape(H // 8, 8, B // 128, 128).transpose(0, 2, 1, 3)
    gather = _make_gather(B, H, V, D, NC, NS)
    out5 = gather(ids4, table)
    return out5.transpose(2, 4, 0, 1, 3).reshape(B, H, D)
